# baseline (speedup 1.0000x reference)
import functools

import jax
import jax.numpy as jnp
import numpy as np
from jax import lax
from jax.experimental import pallas as pl
from jax.experimental.pallas import tpu as pltpu
from jax.experimental.pallas import tpu_sc as plsc

B, D, T = 16, 64, 1024
N = 1024
COMMITMENT_COST = 0.25


_I0 = np.int32(0)


_TT = 1024
_CH = 128


def _tc_body(x_ref, emb_ref, idx_ref, loss_ref):
    x = x_ref[0]
    emb = emb_ref[...]
    b2 = jnp.sum(emb * emb, axis=1)
    a_op = jnp.concatenate([-2.0 * emb, b2[:, None]], axis=1)
    x_op = jnp.concatenate([x, jnp.ones((1, _TT), jnp.float32)], axis=0)
    minv = jnp.full((_TT,), jnp.float32(3.0e38))
    argv = jnp.full((_TT,), jnp.int32(N))
    jloc = lax.broadcasted_iota(jnp.int32, (_CH, _TT), 0)
    for k in range(N // _CH):
        sk = lax.dot_general(
            a_op[k * _CH:(k + 1) * _CH], x_op, (((1,), (0,)), ((), ())),
            preferred_element_type=jnp.float32,
            precision=lax.Precision.HIGHEST)
        lmin = jnp.min(sk, axis=0)
        larg = jnp.min(jnp.where(sk == lmin[None, :], jloc, jnp.int32(N)),
                       axis=0) + jnp.int32(k * _CH)
        better = lmin < minv
        argv = jnp.where(better, larg, argv)
        minv = jnp.where(better, lmin, minv)
    idx_ref[0, 0] = argv
    a2 = jnp.sum(x * x, axis=0)
    part = jnp.sum(minv + a2)

    @pl.when((pl.program_id(0) == 0) & (pl.program_id(1) == 0))
    def _():
        loss_ref[...] = jnp.zeros((1, 1), jnp.float32)

    loss_ref[...] += part.reshape(1, 1)


def _tc_argmin(inputs, embeddings, interpret=False):
    nb = inputs.shape[0]
    return pl.pallas_call(
        _tc_body,
        grid=(nb, T // _TT),
        in_specs=[
            pl.BlockSpec((1, D, _TT), lambda i, j: (i, _I0, j)),
            pl.BlockSpec((N, D), lambda i, j: (_I0, _I0)),
        ],
        out_specs=[
            pl.BlockSpec((1, 1, _TT), lambda i, j: (i, _I0, j)),
            pl.BlockSpec((1, 1), lambda i, j: (_I0, _I0)),
        ],
        out_shape=[
            jax.ShapeDtypeStruct((nb, 1, T), jnp.int32),
            jax.ShapeDtypeStruct((1, 1), jnp.float32),
        ],
        interpret=interpret,
    )(inputs, embeddings)


_NC, _NS, _L = 2, 16, 16
_NW = _NC * _NS
_D_PER_W = D // _NW


def _sc_body(nb, embt_hbm, idx_hbm, out_hbm, rows_v, idx_v, obuf_v, sem):
    wid = lax.axis_index("s") * jnp.int32(_NC) + lax.axis_index("c")
    d0 = wid * jnp.int32(_D_PER_W)
    pltpu.sync_copy(embt_hbm.at[pl.ds(d0 * jnp.int32(N), _D_PER_W * N)],
                    rows_v)
    pltpu.sync_copy(idx_hbm, idx_v)

    copies = []
    for b in range(nb):
        def tile_body(i, _, b=b):
            t0 = pl.multiple_of(i * jnp.int32(_L), _L)
            idx16 = idx_v[pl.ds(t0 + jnp.int32(b * T), _L)]
            for dd in range(_D_PER_W):
                vals = plsc.load_gather(
                    rows_v, [idx16 + jnp.int32(dd * N)])
                obuf_v[pl.ds(t0 + jnp.int32((b * _D_PER_W + dd) * T), _L)] = (
                    vals)
            return jnp.int32(0)

        lax.fori_loop(jnp.int32(0), jnp.int32(T // _L), tile_body, jnp.int32(0))
        copies.append(pltpu.async_copy(
            obuf_v.at[pl.ds(b * _D_PER_W * T, _D_PER_W * T)],
            out_hbm.at[pl.ds((b * jnp.int32(D) + d0) * jnp.int32(T),
                             _D_PER_W * T)],
            sem))
    for c in copies:
        c.wait()


@functools.cache
def _sc_gather_fn(nb):
    return pl.kernel(
        functools.partial(_sc_body, nb),
        out_type=jax.ShapeDtypeStruct((nb * D * T,), jnp.float32),
        mesh=plsc.VectorSubcoreMesh(core_axis_name="c", subcore_axis_name="s"),
        scratch_types=[
            pltpu.VMEM((_D_PER_W * N,), jnp.float32),
            pltpu.VMEM((nb * T,), jnp.int32),
            pltpu.VMEM((nb * _D_PER_W * T,), jnp.float32),
            pltpu.SemaphoreType.DMA,
        ],
        compiler_params=pltpu.CompilerParams(needs_layout_passes=False),
    )


def kernel(inputs, embeddings):
    h = B // 2
    embt = embeddings.T.reshape(-1)
    idx3a, loss_a = _tc_argmin(inputs[:h], embeddings)
    qa = _sc_gather_fn(h)(embt, idx3a.reshape(-1))
    idx3b, loss_b = _tc_argmin(inputs[h:], embeddings)
    qb = _sc_gather_fn(h)(embt, idx3b.reshape(-1))
    idx = jnp.concatenate([idx3a.reshape(h, T), idx3b.reshape(h, T)], axis=0)
    quantized = jnp.concatenate([qa, qb]).reshape(B, D, T)
    loss_sum = loss_a + loss_b
    mse = loss_sum[0, 0] / jnp.float32(B * D * T)
    loss = mse + COMMITMENT_COST * mse
    encoding_indices = idx.reshape(B * T).astype(jnp.int64)
    return (quantized, loss, mse, mse, encoding_indices)


# speedup vs baseline: 1.0429x; 1.0429x over previous
import functools

import jax
import jax.numpy as jnp
import numpy as np
from jax import lax
from jax.experimental import pallas as pl
from jax.experimental.pallas import tpu as pltpu
from jax.experimental.pallas import tpu_sc as plsc

B, D, T = 16, 64, 1024
N = 1024
COMMITMENT_COST = 0.25


_I0 = np.int32(0)


_TT = 2048
_CH = 128


def _tc_body(x_ref, emb_ref, idx_ref, loss_ref):
    x = jnp.concatenate([x_ref[0], x_ref[1]], axis=1)
    emb = emb_ref[...]
    b2 = jnp.sum(emb * emb, axis=1)
    a_op = jnp.concatenate([-2.0 * emb, b2[:, None]], axis=1)
    x_op = jnp.concatenate([x, jnp.ones((1, _TT), jnp.float32)], axis=0)
    minv = jnp.full((_TT,), jnp.float32(3.0e38))
    argv = jnp.full((_TT,), jnp.int32(N))
    jloc = lax.broadcasted_iota(jnp.int32, (_CH, _TT), 0)
    for k in range(N // _CH):
        sk = lax.dot_general(
            a_op[k * _CH:(k + 1) * _CH], x_op, (((1,), (0,)), ((), ())),
            preferred_element_type=jnp.float32,
            precision=lax.Precision.HIGHEST)
        lmin = jnp.min(sk, axis=0)
        larg = jnp.min(jnp.where(sk == lmin[None, :], jloc, jnp.int32(N)),
                       axis=0) + jnp.int32(k * _CH)
        better = lmin < minv
        argv = jnp.where(better, larg, argv)
        minv = jnp.where(better, lmin, minv)
    idx_ref[0, 0] = argv
    a2 = jnp.sum(x * x, axis=0)
    part = jnp.sum(minv + a2)

    @pl.when(pl.program_id(0) == 0)
    def _():
        loss_ref[...] = jnp.zeros((1, 1), jnp.float32)

    loss_ref[...] += part.reshape(1, 1)


def _tc_argmin(inputs, embeddings, interpret=False):
    return pl.pallas_call(
        _tc_body,
        grid=(B // 2,),
        in_specs=[
            pl.BlockSpec((2, D, T), lambda i: (i, _I0, _I0)),
            pl.BlockSpec((N, D), lambda i: (_I0, _I0)),
        ],
        out_specs=[
            pl.BlockSpec((1, 1, _TT), lambda i: (i, _I0, _I0)),
            pl.BlockSpec((1, 1), lambda i: (_I0, _I0)),
        ],
        out_shape=[
            jax.ShapeDtypeStruct((B // 2, 1, _TT), jnp.int32),
            jax.ShapeDtypeStruct((1, 1), jnp.float32),
        ],
        interpret=interpret,
    )(inputs, embeddings)


_NC, _NS, _L = 2, 16, 16
_NW = _NC * _NS
_D_PER_W = D // _NW


def _sc_body(embt_hbm, idx_hbm, out_hbm, rows_v, idx_v, obuf_v, sem):
    wid = lax.axis_index("s") * jnp.int32(_NC) + lax.axis_index("c")
    d0 = wid * jnp.int32(_D_PER_W)
    pltpu.sync_copy(embt_hbm.at[pl.ds(d0 * jnp.int32(N), _D_PER_W * N)],
                    rows_v)
    pltpu.sync_copy(idx_hbm, idx_v)

    copies = []
    for b in range(B):
        def tile_body(i, _, b=b):
            t0 = pl.multiple_of(i * jnp.int32(_L), _L)
            idx16 = idx_v[pl.ds(t0 + jnp.int32(b * T), _L)]
            for dd in range(_D_PER_W):
                vals = plsc.load_gather(
                    rows_v, [idx16 + jnp.int32(dd * N)])
                obuf_v[pl.ds(t0 + jnp.int32((b * _D_PER_W + dd) * T), _L)] = (
                    vals)
            return jnp.int32(0)

        lax.fori_loop(jnp.int32(0), jnp.int32(T // _L), tile_body, jnp.int32(0))
        copies.append(pltpu.async_copy(
            obuf_v.at[pl.ds(b * _D_PER_W * T, _D_PER_W * T)],
            out_hbm.at[pl.ds((b * jnp.int32(D) + d0) * jnp.int32(T),
                             _D_PER_W * T)],
            sem))
    for c in copies:
        c.wait()


@functools.cache
def _sc_gather_fn():
    return pl.kernel(
        _sc_body,
        out_type=jax.ShapeDtypeStruct((B * D * T,), jnp.float32),
        mesh=plsc.VectorSubcoreMesh(core_axis_name="c", subcore_axis_name="s"),
        scratch_types=[
            pltpu.VMEM((_D_PER_W * N,), jnp.float32),
            pltpu.VMEM((B * T,), jnp.int32),
            pltpu.VMEM((B * _D_PER_W * T,), jnp.float32),
            pltpu.SemaphoreType.DMA,
        ],
        compiler_params=pltpu.CompilerParams(needs_layout_passes=False),
    )


def kernel(inputs, embeddings):
    idx3, loss_sum = _tc_argmin(inputs, embeddings)
    idx = idx3.reshape(B, T)
    quantized = _sc_gather_fn()(
        embeddings.T.reshape(-1), idx.reshape(-1)).reshape(B, D, T)
    mse = loss_sum[0, 0] / jnp.float32(B * D * T)
    loss = mse + COMMITMENT_COST * mse
    encoding_indices = idx.reshape(B * T).astype(jnp.int64)
    return (quantized, loss, mse, mse, encoding_indices)
